# baseline (speedup 1.0000x reference)
import functools

import numpy as np
import jax
import jax.numpy as jnp
from jax import lax
from jax.experimental import pallas as pl
from jax.experimental.pallas import tpu as pltpu
from jax.experimental.pallas import tpu_sc as plsc

_T, _B, _C = 4096, 16, 192
_VIS_T = _T - int(_T * 0.75)
_NC, _NS = 2, 16
_NW = _NC * _NS
_CT = _C // 8
_CTW = _CT // 2
_LANES = 16


@functools.cache
def _host_indexes():
    with jax.default_device(jax.local_devices(backend="cpu")[0]):
        base = jax.random.key(42)
        perms = [jax.random.permutation(jax.random.fold_in(base, b), _T)
                 for b in range(_B)]
        fwd = np.asarray(jnp.stack(perms, axis=-1).astype(jnp.int32))
    bwd = np.argsort(fwd, axis=0).astype(np.int32)
    gidx = np.stack([fwd[:_VIS_T, w // 2].reshape(8, 128)
                     for w in range(_NW)]).astype(np.int32)
    return fwd, bwd, gidx


_FWD_NP, _BWD_NP, _GIDX_NP = _host_indexes()


@functools.cache
def _build_gather():
    @functools.partial(
        pl.kernel,
        mesh=plsc.VectorSubcoreMesh(core_axis_name="c", subcore_axis_name="s"),
        compiler_params=pltpu.CompilerParams(use_tc_tiling_on_sc=True,
                                             needs_layout_passes=False,
                                             disable_bounds_checks=True,
                                             disable_semaphore_checks=True,
                                             skip_device_barrier=True),
        out_type=(
            jax.ShapeDtypeStruct((_B, _C, _VIS_T), jnp.float32),
            jax.ShapeDtypeStruct((_B, _T), jnp.int32),
            jax.ShapeDtypeStruct((_B, _T), jnp.int32),
        ),
        scratch_types=[
            pltpu.VMEM((8, 128), jnp.int32),
            pltpu.VMEM((8, _T), jnp.float32),
            pltpu.VMEM((8, _T), jnp.float32),
            pltpu.VMEM((8, _T), jnp.float32),
            pltpu.VMEM((8, _VIS_T), jnp.float32),
            pltpu.VMEM((8, _VIS_T), jnp.float32),
            pltpu.VMEM((8, _VIS_T), jnp.float32),
            pltpu.VMEM((2, 8, 256), jnp.int32),
            pltpu.SemaphoreType.DMA,
            pltpu.SemaphoreType.DMA,
            pltpu.SemaphoreType.DMA,
            pltpu.SemaphoreType.DMA,
            pltpu.SemaphoreType.DMA,
            pltpu.SemaphoreType.DMA,
            pltpu.SemaphoreType.DMA,
            pltpu.SemaphoreType.DMA,
            pltpu.SemaphoreType.DMA,
        ],
    )
    def _gather(tbl_hbm, fwd_hbm, bwd_hbm, gidx_hbm,
                vis_hbm, fwd_out, bwd_out,
                idx_v, in_0, in_1, in_2, out_0, out_1, out_2, pf_v,
                sem_0, sem_1, sem_2, sem_o0, sem_o1, sem_o2,
                sem_ix, sem_pf, sem_pb):
        wid = lax.axis_index("s") * _NC + lax.axis_index("c")
        b = wid // 2
        base = (wid % 2) * _CTW

        lix = pltpu.async_copy(gidx_hbm.at[wid], idx_v, sem_ix)

        def in_slab(ct):
            return tbl_hbm.at[b, pl.ds(ct * 8, 8), :]

        def out_slab(ct):
            return vis_hbm.at[b, pl.ds(ct * 8, 8), :]

        ins = ((in_0, sem_0), (in_1, sem_1), (in_2, sem_2))
        outs = ((out_0, sem_o0), (out_1, sem_o1), (out_2, sem_o2))

        for j in range(3):
            pltpu.async_copy(in_slab(base + j), ins[j][0], ins[j][1])

        r0 = (wid % 2) * 8
        c0 = (wid // 2) * 256
        lpf = pltpu.async_copy(fwd_hbm.at[pl.ds(r0, 8), pl.ds(c0, 256)],
                               pf_v.at[0], sem_pf)
        lpb = pltpu.async_copy(bwd_hbm.at[pl.ds(r0, 8), pl.ds(c0, 256)],
                               pf_v.at[1], sem_pb)
        lix.wait()

        def compute(in_v, out_v):
            for k in range(_VIS_T // _LANES):
                tv = idx_v[k // 8, pl.ds((k % 8) * _LANES, _LANES)]
                for r in range(8):
                    rv = jnp.full((_LANES,), r, jnp.int32)
                    out_v[r, pl.ds(k * _LANES, _LANES)] = plsc.load_gather(
                        in_v, [rv, tv])

        def step(m, ct, j):
            in_v, sem_i = ins[j]
            out_v, sem_o = outs[j]
            pltpu.make_async_copy(in_slab(ct), in_v, sem_i).wait()

            @pl.when(m > 0)
            def _():
                pltpu.make_async_copy(out_v, out_slab(ct - 3), sem_o).wait()

            compute(in_v, out_v)
            pltpu.async_copy(out_v, out_slab(ct), sem_o)

            @pl.when(ct + 3 < base + _CTW)
            def _():
                pltpu.async_copy(in_slab(ct + 3), in_v, sem_i)

        def body(m, carry):
            for j in range(3):
                step(m, base + 3 * m + j, j)
            return carry

        lax.fori_loop(0, _CTW // 3, body, 0)
        lpf.wait()
        pltpu.async_copy(pf_v.at[0], fwd_out.at[pl.ds(r0, 8), pl.ds(c0, 256)],
                         sem_pf)
        lpb.wait()
        pltpu.async_copy(pf_v.at[1], bwd_out.at[pl.ds(r0, 8), pl.ds(c0, 256)],
                         sem_pb)
        pltpu.make_async_copy(out_0, out_slab(base + _CTW - 3), sem_o0).wait()
        pltpu.make_async_copy(out_1, out_slab(base + _CTW - 2), sem_o1).wait()
        pltpu.make_async_copy(out_2, out_slab(base + _CTW - 1), sem_o2).wait()
        pltpu.make_async_copy(pf_v.at[0],
                              fwd_out.at[pl.ds(r0, 8), pl.ds(c0, 256)],
                              sem_pf).wait()
        pltpu.make_async_copy(pf_v.at[1],
                              bwd_out.at[pl.ds(r0, 8), pl.ds(c0, 256)],
                              sem_pb).wait()

    return _gather


def kernel(patches):
    tblT = jnp.transpose(patches, (1, 2, 0))
    visT, fwdT, bwdT = _build_gather()(
        tblT, jnp.asarray(_FWD_NP.T), jnp.asarray(_BWD_NP.T),
        jnp.asarray(_GIDX_NP))
    vis = jnp.transpose(visT, (2, 0, 1))
    return (vis, fwdT.T, bwdT.T, jnp.int32(_VIS_T))


# speedup vs baseline: 1.1970x; 1.1970x over previous
import functools

import numpy as np
import jax
import jax.numpy as jnp
from jax import lax
from jax.experimental import pallas as pl
from jax.experimental.pallas import tpu as pltpu
from jax.experimental.pallas import tpu_sc as plsc

_T, _B, _C = 4096, 16, 192
_VIS_T = _T - int(_T * 0.75)
_NC, _NS = 2, 16
_NW = _NC * _NS
_CT = _C // 8
_CTW = _CT // 2
_LANES = 16


@functools.cache
def _host_indexes():
    with jax.default_device(jax.local_devices(backend="cpu")[0]):
        base = jax.random.key(42)
        perms = [jax.random.permutation(jax.random.fold_in(base, b), _T)
                 for b in range(_B)]
        fwd = np.asarray(jnp.stack(perms, axis=-1).astype(jnp.int32))
    bwd = np.argsort(fwd, axis=0).astype(np.int32)
    gidx = np.stack([fwd[:_VIS_T, w // 2].reshape(8, 128)
                     for w in range(_NW)]).astype(np.int32)
    return fwd, bwd, gidx


_FWD_NP, _BWD_NP, _GIDX_NP = _host_indexes()


@functools.cache
def _build_gather():
    @functools.partial(
        pl.kernel,
        mesh=plsc.VectorSubcoreMesh(core_axis_name="c", subcore_axis_name="s"),
        compiler_params=pltpu.CompilerParams(use_tc_tiling_on_sc=True,
                                             needs_layout_passes=False,
                                             disable_bounds_checks=True,
                                             disable_semaphore_checks=True,
                                             skip_device_barrier=True),
        out_type=(
            jax.ShapeDtypeStruct((_B, _C, _VIS_T), jnp.float32),
            jax.ShapeDtypeStruct((_B, _T), jnp.int32),
            jax.ShapeDtypeStruct((_B, _T), jnp.int32),
        ),
        scratch_types=[
            pltpu.VMEM((8, 128), jnp.int32),
            pltpu.VMEM((8, _T), jnp.float32),
            pltpu.VMEM((8, _T), jnp.float32),
            pltpu.VMEM((8, _T), jnp.float32),
            pltpu.VMEM((8, _VIS_T), jnp.float32),
            pltpu.VMEM((8, _VIS_T), jnp.float32),
            pltpu.VMEM((2, 8, 256), jnp.int32),
            pltpu.SemaphoreType.DMA,
            pltpu.SemaphoreType.DMA,
            pltpu.SemaphoreType.DMA,
            pltpu.SemaphoreType.DMA,
            pltpu.SemaphoreType.DMA,
            pltpu.SemaphoreType.DMA,
            pltpu.SemaphoreType.DMA,
            pltpu.SemaphoreType.DMA,
        ],
    )
    def _gather(tbl_hbm, fwd_hbm, bwd_hbm, gidx_hbm,
                vis_hbm, fwd_out, bwd_out,
                idx_v, in_0, in_1, in_2, out_0, out_1, pf_v,
                sem_0, sem_1, sem_2, sem_o0, sem_o1,
                sem_ix, sem_pf, sem_pb):
        wid = lax.axis_index("s") * _NC + lax.axis_index("c")
        b = wid // 2
        base = (wid % 2) * _CTW

        lix = pltpu.async_copy(gidx_hbm.at[wid], idx_v, sem_ix)

        def in_slab(ct):
            return tbl_hbm.at[b, pl.ds(ct * 8, 8), :]

        def out_slab(ct):
            return vis_hbm.at[b, pl.ds(ct * 8, 8), :]

        ins = ((in_0, sem_0), (in_1, sem_1), (in_2, sem_2))
        outs = ((out_0, sem_o0), (out_1, sem_o1))

        for j in range(3):
            pltpu.async_copy(in_slab(base + j), ins[j][0], ins[j][1])

        r0 = (wid % 2) * 8
        c0 = (wid // 2) * 256
        lpf = pltpu.async_copy(fwd_hbm.at[pl.ds(r0, 8), pl.ds(c0, 256)],
                               pf_v.at[0], sem_pf)
        lpb = pltpu.async_copy(bwd_hbm.at[pl.ds(r0, 8), pl.ds(c0, 256)],
                               pf_v.at[1], sem_pb)
        lix.wait()

        def compute(in_v, out_v):
            def kbody(k, carry):
                tv = idx_v[k // 8, pl.ds((k % 8) * _LANES, _LANES)]
                for r in range(8):
                    rv = jnp.full((_LANES,), r, jnp.int32)
                    out_v[r, pl.ds(k * _LANES, _LANES)] = plsc.load_gather(
                        in_v, [rv, tv])
                return carry

            lax.fori_loop(0, _VIS_T // _LANES, kbody, 0)

        def step(s, ct):
            in_v, sem_i = ins[s % 3]
            out_v, sem_o = outs[s % 2]
            pltpu.make_async_copy(in_slab(ct), in_v, sem_i).wait()

            if s >= 2:
                pltpu.make_async_copy(out_v, out_slab(ct - 2), sem_o).wait()

            compute(in_v, out_v)
            pltpu.async_copy(out_v, out_slab(ct), sem_o)

            if s + 3 < _CTW:
                pltpu.async_copy(in_slab(ct + 3), in_v, sem_i)

        for s in range(_CTW):
            step(s, base + s)
        lpf.wait()
        pltpu.async_copy(pf_v.at[0], fwd_out.at[pl.ds(r0, 8), pl.ds(c0, 256)],
                         sem_pf)
        lpb.wait()
        pltpu.async_copy(pf_v.at[1], bwd_out.at[pl.ds(r0, 8), pl.ds(c0, 256)],
                         sem_pb)
        pltpu.make_async_copy(out_0, out_slab(base + _CTW - 2), sem_o0).wait()
        pltpu.make_async_copy(out_1, out_slab(base + _CTW - 1), sem_o1).wait()
        pltpu.make_async_copy(pf_v.at[0],
                              fwd_out.at[pl.ds(r0, 8), pl.ds(c0, 256)],
                              sem_pf).wait()
        pltpu.make_async_copy(pf_v.at[1],
                              bwd_out.at[pl.ds(r0, 8), pl.ds(c0, 256)],
                              sem_pb).wait()

    return _gather


def kernel(patches):
    tblT = jnp.transpose(patches, (1, 2, 0))
    visT, fwdT, bwdT = _build_gather()(
        tblT, jnp.asarray(_FWD_NP.T), jnp.asarray(_BWD_NP.T),
        jnp.asarray(_GIDX_NP))
    vis = jnp.transpose(visT, (2, 0, 1))
    return (vis, fwdT.T, bwdT.T, jnp.int32(_VIS_T))
